# baseline (speedup 1.0000x reference)
import jax
import jax.numpy as jnp
from jax import lax
from jax.experimental import pallas as pl
from jax.experimental.pallas import tpu as pltpu
from jax.experimental.pallas import tpu_sc as plsc

N = 50000
NFEAT = 256
NHID = 256
F = 64
E = 800000
K = 10
ALPHA = 0.1

NC = 2
NS = 16

HALF = 25088
NPAD = 2 * HALF
TRASH = HALF
AGG_ROWS = HALF + 8

ROWS_PER_TILE = HALF // NS
RBLK = 28
NBLK = ROWS_PER_TILE // RBLK

EROWS = 6272
EPAD = EROWS * 128
ROWS_PER_TILE_E = EROWS // NS
MACROS = ROWS_PER_TILE_E // 8

PROWS = 420
SMAC = 14

_mesh = plsc.VectorSubcoreMesh(core_axis_name="c", subcore_axis_name="s",
                               num_cores=NC, num_subcores=NS)
_sc_params = pltpu.CompilerParams(use_tc_tiling_on_sc=False)
_sc_params_nl = pltpu.CompilerParams(use_tc_tiling_on_sc=False,
                                     needs_layout_passes=False)


def _mlp_body(x_ref, w1_ref, b1_ref, w2_ref, b2_ref, o_ref):
    h = jnp.dot(x_ref[...], w1_ref[...], preferred_element_type=jnp.float32)
    h = jnp.maximum(h + b1_ref[...], 0.0)
    o_ref[...] = (
        jnp.dot(h, w2_ref[...], preferred_element_type=jnp.float32)
        + b2_ref[...]
    )


def _mlp(xp, W1, b1, W2, b2):
    blk = 512
    grid = NPAD // blk
    return pl.pallas_call(
        _mlp_body,
        grid=(grid,),
        in_specs=[
            pl.BlockSpec((blk, NFEAT), lambda i: (i, 0)),
            pl.BlockSpec((NFEAT, NHID), lambda i: (0, 0)),
            pl.BlockSpec((1, NHID), lambda i: (0, 0)),
            pl.BlockSpec((NHID, F), lambda i: (0, 0)),
            pl.BlockSpec((1, F), lambda i: (0, 0)),
        ],
        out_specs=pl.BlockSpec((blk, F), lambda i: (i, 0)),
        out_shape=jax.ShapeDtypeStruct((NPAD, F), jnp.float32),
    )(xp, W1, b1.reshape(1, NHID), W2, b2.reshape(1, F))


def _prep_body(src_hbm, dst_hbm, srcp_hbm, selp_hbm, cnt_hbm, deg_hbm,
               dst_v, srcv_v, sel_v, ones_v, degbuf_v, sts_v, stl_v, cnt_v,
               deg_sh):
    cid = lax.axis_index("c")
    sid = lax.axis_index("s")

    @pl.loop(0, ROWS_PER_TILE)
    def _zero(i):
        degbuf_v[i, :] = jnp.zeros((16,), jnp.float32)
    pltpu.sync_copy(degbuf_v,
                    deg_sh.at[pl.ds(sid * ROWS_PER_TILE, ROWS_PER_TILE)])

    @pl.loop(0, 128)
    def _ones(i):
        ones_v[i, :] = jnp.ones((16,), jnp.float32)

    plsc.subcore_barrier()

    lo = cid * HALF
    flat = srcp_hbm

    @pl.loop(0, MACROS, init_carry=(jnp.int32(0), jnp.int32(0)))
    def _macro(g, carry):
        off, rows = carry
        r0 = sid * ROWS_PER_TILE_E + g * 8
        pltpu.sync_copy(dst_hbm.at[pl.ds(r0, 8)], dst_v)
        pltpu.sync_copy(src_hbm.at[pl.ds(r0, 8)], srcv_v)
        for j in range(8):
            for q in range(8):
                sl = pl.ds(q * 16, 16)
                d = dst_v[j, sl]
                s = srcv_v[j, sl]
                loc = d - lo
                ok = (d >= lo) & (d < lo + HALF)
                sel_v[j, sl] = jnp.where(ok, loc, TRASH)
                inc = jnp.where(ok, jnp.int32(1), jnp.int32(0))
                cum = lax.cumsum(inc, axis=0)
                pos = jnp.where(ok, off + cum - 1, jnp.int32(1264))
                plsc.store_scatter(sts_v, [pos], s)
                plsc.store_scatter(stl_v, [pos], loc)
                off = off + cum[15]
                do_flush = off >= 1024

                @pl.when(do_flush)
                def _flush():
                    pltpu.sync_copy(
                        sts_v.at[pl.ds(0, 1024)],
                        srcp_hbm.at[cid, sid, pl.ds(rows * 128, 1024)])
                    pltpu.sync_copy(
                        stl_v.at[pl.ds(0, 1024)],
                        selp_hbm.at[cid, sid, pl.ds(rows * 128, 1024)])
                    sts_v[pl.ds(0, 16)] = sts_v[pl.ds(1024, 16)]
                    stl_v[pl.ds(0, 16)] = stl_v[pl.ds(1024, 16)]

                off = jnp.where(do_flush, off - 1024, off)
                rows = jnp.where(do_flush, rows + 8, rows)
        for j in range(8):
            pltpu.sync_copy(ones_v, deg_sh.at[sel_v.at[j]], add=True)
        return off, rows

    off, rows = _macro
    pad_s = jnp.zeros((16,), jnp.int32)
    pad_l = jnp.full((16,), TRASH, jnp.int32)
    sts_v[pl.ds(off, 16)] = pad_s
    stl_v[pl.ds(off, 16)] = pad_l
    target = ((off + 127) // 128) * 128
    for k in range(7):
        pos = off + 16 + k * 16

        @pl.when(pos < target)
        def _pad():
            sts_v[pl.ds(pos, 16)] = pad_s
            stl_v[pl.ds(pos, 16)] = pad_l

    @pl.when(target > 0)
    def _final_flush():
        pltpu.sync_copy(sts_v.at[pl.ds(0, 1152)],
                        srcp_hbm.at[cid, sid, pl.ds(rows * 128, 1152)])
        pltpu.sync_copy(stl_v.at[pl.ds(0, 1152)],
                        selp_hbm.at[cid, sid, pl.ds(rows * 128, 1152)])

    nrows = rows + target // 128
    cnt_v[...] = jnp.full((16,), nrows, jnp.int32)
    pltpu.sync_copy(cnt_v, cnt_hbm.at[cid, sid])

    plsc.subcore_barrier()

    pltpu.sync_copy(deg_sh.at[pl.ds(sid * ROWS_PER_TILE, ROWS_PER_TILE)],
                    deg_hbm.at[pl.ds(cid * HALF + sid * ROWS_PER_TILE,
                                     ROWS_PER_TILE)])


def _prep(src128, dst128):
    return pl.kernel(
        _prep_body,
        out_type=(
            jax.ShapeDtypeStruct((NC, NS, PROWS * 128), jnp.int32),
            jax.ShapeDtypeStruct((NC, NS, PROWS * 128), jnp.int32),
            jax.ShapeDtypeStruct((NC, NS, 16), jnp.int32),
            jax.ShapeDtypeStruct((NPAD, 16), jnp.float32),
        ),
        mesh=_mesh,
        scratch_types=[
            pltpu.VMEM((8, 128), jnp.int32),
            pltpu.VMEM((8, 128), jnp.int32),
            pltpu.VMEM((8, 128), jnp.int32),
            pltpu.VMEM((128, 16), jnp.float32),
            pltpu.VMEM((ROWS_PER_TILE, 16), jnp.float32),
            pltpu.VMEM((1280,), jnp.int32),
            pltpu.VMEM((1280,), jnp.int32),
            pltpu.VMEM((16,), jnp.int32),
            pltpu.VMEM_SHARED((AGG_ROWS, 16), jnp.float32),
        ],
        compiler_params=_sc_params_nl,
    )(src128, dst128)


def _coeff_body(deg_ref, h_ref, u_ref, ca1_ref, ca2_ref):
    dinv = lax.rsqrt(deg_ref[:, :1] + 1.0)
    h = h_ref[...]
    u = dinv * h
    u_ref[...] = u
    ca1_ref[...] = jnp.concatenate(
        [jnp.broadcast_to((1.0 - ALPHA) * dinv * dinv, h.shape), ALPHA * u],
        axis=1)
    ca2_ref[...] = jnp.concatenate(
        [jnp.broadcast_to((1.0 - ALPHA) * dinv, h.shape), ALPHA * h], axis=1)


def _coeff(deg, h):
    blk = 512
    grid = NPAD // blk
    o = jax.ShapeDtypeStruct((NPAD, F), jnp.float32)
    o2 = jax.ShapeDtypeStruct((NPAD, 2 * F), jnp.float32)
    return pl.pallas_call(
        _coeff_body,
        grid=(grid,),
        in_specs=[
            pl.BlockSpec((blk, 16), lambda i: (i, 0)),
            pl.BlockSpec((blk, F), lambda i: (i, 0)),
        ],
        out_specs=[
            pl.BlockSpec((blk, F), lambda i: (i, 0)),
            pl.BlockSpec((blk, 2 * F), lambda i: (i, 0)),
            pl.BlockSpec((blk, 2 * F), lambda i: (i, 0)),
        ],
        out_shape=(o, o2, o2),
    )(deg, h)


def _step_body(u_hbm, srcp_hbm, selp_hbm, cnt_hbm, ca_hbm, out_hbm,
               src_v, sel_v, rows_v, aggb_v, cab_v, cnt_v, agg_sh,
               gsem, ssem):
    cid = lax.axis_index("c")
    sid = lax.axis_index("s")

    l0 = sid * ROWS_PER_TILE
    pltpu.sync_copy(u_hbm.at[pl.ds(cid * HALF + l0, ROWS_PER_TILE)],
                    agg_sh.at[pl.ds(l0, ROWS_PER_TILE)])
    pltpu.sync_copy(cnt_hbm.at[cid, sid], cnt_v)
    nb = cnt_v[pl.ds(0, 16)][0]

    plsc.subcore_barrier()

    def _stage(m, slot):
        pltpu.sync_copy(srcp_hbm.at[cid, sid, pl.ds(m * SMAC, SMAC)],
                        src_v.at[slot])
        pltpu.sync_copy(selp_hbm.at[cid, sid, pl.ds(m * SMAC, SMAC)],
                        sel_v.at[slot])

    def _gather(t, b):
        m = t // SMAC
        pltpu.async_copy(u_hbm.at[src_v.at[m % 2, t % SMAC]], rows_v.at[b],
                         gsem)

    @pl.when(nb > 0)
    def _():
        _stage(0, 0)
        _gather(0, 0)

    @pl.when(nb > 1)
    def _():
        _gather(1, 1)

    @pl.loop(0, nb)
    def _edge(t):
        b = t % 2
        m = t // SMAC
        j = t % SMAC
        pltpu.make_async_copy(u_hbm.at[src_v.at[m % 2, j]],
                              rows_v.at[b], gsem).wait()
        pltpu.async_copy(rows_v.at[b], agg_sh.at[sel_v.at[m % 2, j]],
                         ssem, add=True)

        @pl.when(jnp.logical_and(j == SMAC - 2, (m + 1) * SMAC < nb))
        def _():
            _stage(m + 1, (m + 1) % 2)

        pltpu.make_async_copy(rows_v.at[b], agg_sh.at[sel_v.at[m % 2, j]],
                              ssem).wait()

        @pl.when(t + 2 < nb)
        def _():
            _gather(t + 2, b)

    plsc.subcore_barrier()

    @pl.loop(0, NBLK)
    def _combine(i):
        lb = sid * ROWS_PER_TILE + i * RBLK
        g0 = cid * HALF + lb
        pltpu.sync_copy(agg_sh.at[pl.ds(lb, RBLK)], aggb_v)
        pltpu.sync_copy(ca_hbm.at[pl.ds(g0, RBLK)], cab_v)

        @pl.loop(0, RBLK)
        def _row(r):
            for q in range(F // 16):
                sl = pl.ds(q * 16, 16)
                aggb_v[r, sl] = (aggb_v[r, sl] * cab_v[r, sl]
                                 + cab_v[r, pl.ds(F + q * 16, 16)])

        pltpu.sync_copy(aggb_v, out_hbm.at[pl.ds(g0, RBLK)])


def _step(u, srcp, selp, cnt, ca):
    return pl.kernel(
        _step_body,
        out_type=jax.ShapeDtypeStruct((NPAD, F), jnp.float32),
        mesh=_mesh,
        scratch_types=[
            pltpu.VMEM((2, SMAC, 128), jnp.int32),
            pltpu.VMEM((2, SMAC, 128), jnp.int32),
            pltpu.VMEM((2, 128, F), jnp.float32),
            pltpu.VMEM((RBLK, F), jnp.float32),
            pltpu.VMEM((RBLK, 2 * F), jnp.float32),
            pltpu.VMEM((16,), jnp.int32),
            pltpu.VMEM_SHARED((AGG_ROWS, F), jnp.float32),
            pltpu.SemaphoreType.DMA,
            pltpu.SemaphoreType.DMA,
        ],
        compiler_params=_sc_params,
    )(u, srcp, selp, cnt, ca)


def kernel(x, edge_index, W1, b1, W2, b2):
    xp = jnp.pad(x, ((0, NPAD - N), (0, 0)))
    h = _mlp(xp, W1, b1, W2, b2)

    src = jnp.pad(edge_index[0], (0, EPAD - E)).reshape(EROWS, 128)
    dst = jnp.pad(edge_index[1], (0, EPAD - E),
                  constant_values=2 ** 20).reshape(EROWS, 128)

    srcp, selp, cnt, deg = _prep(src, dst)
    srcp = srcp.reshape(NC, NS, PROWS, 128)
    selp = selp.reshape(NC, NS, PROWS, 128)
    u, ca1, ca2 = _coeff(deg, h)

    for _ in range(K - 1):
        u = _step(u, srcp, selp, cnt, ca1)
    z = _step(u, srcp, selp, cnt, ca2)
    return z[:N]


# speedup vs baseline: 1.0494x; 1.0494x over previous
import jax
import jax.numpy as jnp
from jax import lax
from jax.experimental import pallas as pl
from jax.experimental.pallas import tpu as pltpu
from jax.experimental.pallas import tpu_sc as plsc

N = 50000
NFEAT = 256
NHID = 256
F = 64
E = 800000
K = 10
ALPHA = 0.1

NC = 2
NS = 16

HALF = 25088
NPAD = 2 * HALF
TRASH = HALF
AGG_ROWS = HALF + 8

ROWS_PER_TILE = HALF // NS
RBLK = 56
NBLK = ROWS_PER_TILE // RBLK

EROWS = 6272
EPAD = EROWS * 128
ROWS_PER_TILE_E = EROWS // NS
MACROS = ROWS_PER_TILE_E // 8

PROWS = 420
SMAC = 14

_mesh = plsc.VectorSubcoreMesh(core_axis_name="c", subcore_axis_name="s",
                               num_cores=NC, num_subcores=NS)
_sc_params = pltpu.CompilerParams(use_tc_tiling_on_sc=False)
_sc_params_nl = pltpu.CompilerParams(use_tc_tiling_on_sc=False,
                                     needs_layout_passes=False)


def _mlp_body(x_ref, w1_ref, b1_ref, w2_ref, b2_ref, o_ref):
    h = jnp.dot(x_ref[...], w1_ref[...], preferred_element_type=jnp.float32)
    h = jnp.maximum(h + b1_ref[...], 0.0)
    o_ref[...] = (
        jnp.dot(h, w2_ref[...], preferred_element_type=jnp.float32)
        + b2_ref[...]
    )


def _mlp(xp, W1, b1, W2, b2):
    blk = 512
    grid = NPAD // blk
    return pl.pallas_call(
        _mlp_body,
        grid=(grid,),
        in_specs=[
            pl.BlockSpec((blk, NFEAT), lambda i: (i, 0)),
            pl.BlockSpec((NFEAT, NHID), lambda i: (0, 0)),
            pl.BlockSpec((1, NHID), lambda i: (0, 0)),
            pl.BlockSpec((NHID, F), lambda i: (0, 0)),
            pl.BlockSpec((1, F), lambda i: (0, 0)),
        ],
        out_specs=pl.BlockSpec((blk, F), lambda i: (i, 0)),
        out_shape=jax.ShapeDtypeStruct((NPAD, F), jnp.float32),
    )(xp, W1, b1.reshape(1, NHID), W2, b2.reshape(1, F))


def _prep_body(src_hbm, dst_hbm, srcp_hbm, selp_hbm, cnt_hbm, deg_hbm,
               dst_v, srcv_v, sel_v, ones_v, degbuf_v, sts_v, stl_v, cnt_v,
               deg_sh):
    cid = lax.axis_index("c")
    sid = lax.axis_index("s")

    @pl.loop(0, ROWS_PER_TILE)
    def _zero(i):
        degbuf_v[i, :] = jnp.zeros((16,), jnp.float32)
    pltpu.sync_copy(degbuf_v,
                    deg_sh.at[pl.ds(sid * ROWS_PER_TILE, ROWS_PER_TILE)])

    @pl.loop(0, 128)
    def _ones(i):
        ones_v[i, :] = jnp.ones((16,), jnp.float32)

    plsc.subcore_barrier()

    lo = cid * HALF
    flat = srcp_hbm

    @pl.loop(0, MACROS, init_carry=(jnp.int32(0), jnp.int32(0)))
    def _macro(g, carry):
        off, rows = carry
        r0 = sid * ROWS_PER_TILE_E + g * 8
        pltpu.sync_copy(dst_hbm.at[pl.ds(r0, 8)], dst_v)
        pltpu.sync_copy(src_hbm.at[pl.ds(r0, 8)], srcv_v)
        for j in range(8):
            for q in range(8):
                sl = pl.ds(q * 16, 16)
                d = dst_v[j, sl]
                s = srcv_v[j, sl]
                loc = d - lo
                ok = (d >= lo) & (d < lo + HALF)
                sel_v[j, sl] = jnp.where(ok, loc, TRASH)
                inc = jnp.where(ok, jnp.int32(1), jnp.int32(0))
                cum = lax.cumsum(inc, axis=0)
                pos = jnp.where(ok, off + cum - 1, jnp.int32(1264))
                plsc.store_scatter(sts_v, [pos], s)
                plsc.store_scatter(stl_v, [pos], loc)
                off = off + cum[15]
                do_flush = off >= 1024

                @pl.when(do_flush)
                def _flush():
                    pltpu.sync_copy(
                        sts_v.at[pl.ds(0, 1024)],
                        srcp_hbm.at[cid, sid, pl.ds(rows * 128, 1024)])
                    pltpu.sync_copy(
                        stl_v.at[pl.ds(0, 1024)],
                        selp_hbm.at[cid, sid, pl.ds(rows * 128, 1024)])
                    sts_v[pl.ds(0, 16)] = sts_v[pl.ds(1024, 16)]
                    stl_v[pl.ds(0, 16)] = stl_v[pl.ds(1024, 16)]

                off = jnp.where(do_flush, off - 1024, off)
                rows = jnp.where(do_flush, rows + 8, rows)
        for j in range(8):
            pltpu.sync_copy(ones_v, deg_sh.at[sel_v.at[j]], add=True)
        return off, rows

    off, rows = _macro
    pad_s = jnp.zeros((16,), jnp.int32)
    pad_l = jnp.full((16,), TRASH, jnp.int32)
    sts_v[pl.ds(off, 16)] = pad_s
    stl_v[pl.ds(off, 16)] = pad_l
    target = ((off + 127) // 128) * 128
    for k in range(7):
        pos = off + 16 + k * 16

        @pl.when(pos < target)
        def _pad():
            sts_v[pl.ds(pos, 16)] = pad_s
            stl_v[pl.ds(pos, 16)] = pad_l

    @pl.when(target > 0)
    def _final_flush():
        pltpu.sync_copy(sts_v.at[pl.ds(0, 1152)],
                        srcp_hbm.at[cid, sid, pl.ds(rows * 128, 1152)])
        pltpu.sync_copy(stl_v.at[pl.ds(0, 1152)],
                        selp_hbm.at[cid, sid, pl.ds(rows * 128, 1152)])

    nrows = rows + target // 128
    cnt_v[...] = jnp.full((16,), nrows, jnp.int32)
    pltpu.sync_copy(cnt_v, cnt_hbm.at[cid, sid])

    plsc.subcore_barrier()

    pltpu.sync_copy(deg_sh.at[pl.ds(sid * ROWS_PER_TILE, ROWS_PER_TILE)],
                    deg_hbm.at[pl.ds(cid * HALF + sid * ROWS_PER_TILE,
                                     ROWS_PER_TILE)])


def _prep(src128, dst128):
    return pl.kernel(
        _prep_body,
        out_type=(
            jax.ShapeDtypeStruct((NC, NS, PROWS * 128), jnp.int32),
            jax.ShapeDtypeStruct((NC, NS, PROWS * 128), jnp.int32),
            jax.ShapeDtypeStruct((NC, NS, 16), jnp.int32),
            jax.ShapeDtypeStruct((NPAD, 16), jnp.float32),
        ),
        mesh=_mesh,
        scratch_types=[
            pltpu.VMEM((8, 128), jnp.int32),
            pltpu.VMEM((8, 128), jnp.int32),
            pltpu.VMEM((8, 128), jnp.int32),
            pltpu.VMEM((128, 16), jnp.float32),
            pltpu.VMEM((ROWS_PER_TILE, 16), jnp.float32),
            pltpu.VMEM((1280,), jnp.int32),
            pltpu.VMEM((1280,), jnp.int32),
            pltpu.VMEM((16,), jnp.int32),
            pltpu.VMEM_SHARED((AGG_ROWS, 16), jnp.float32),
        ],
        compiler_params=_sc_params_nl,
    )(src128, dst128)


def _coeff_body(deg_ref, h_ref, u_ref, ca1_ref, ca2_ref):
    dinv = lax.rsqrt(deg_ref[:, :1] + 1.0)
    h = h_ref[...]
    u = dinv * h
    u_ref[...] = u
    ca1_ref[...] = jnp.concatenate(
        [jnp.broadcast_to((1.0 - ALPHA) * dinv * dinv, h.shape), ALPHA * u],
        axis=1)
    ca2_ref[...] = jnp.concatenate(
        [jnp.broadcast_to((1.0 - ALPHA) * dinv, h.shape), ALPHA * h], axis=1)


def _coeff(deg, h):
    blk = 512
    grid = NPAD // blk
    o = jax.ShapeDtypeStruct((NPAD, F), jnp.float32)
    o2 = jax.ShapeDtypeStruct((NPAD, 2 * F), jnp.float32)
    return pl.pallas_call(
        _coeff_body,
        grid=(grid,),
        in_specs=[
            pl.BlockSpec((blk, 16), lambda i: (i, 0)),
            pl.BlockSpec((blk, F), lambda i: (i, 0)),
        ],
        out_specs=[
            pl.BlockSpec((blk, F), lambda i: (i, 0)),
            pl.BlockSpec((blk, 2 * F), lambda i: (i, 0)),
            pl.BlockSpec((blk, 2 * F), lambda i: (i, 0)),
        ],
        out_shape=(o, o2, o2),
    )(deg, h)


def _step_body(u_hbm, srcp_hbm, selp_hbm, cnt_hbm, ca_hbm, out_hbm,
               cnt_v, agg_sh, gsem, ssem, csem, wsem):
    cid = lax.axis_index("c")
    sid = lax.axis_index("s")

    l0 = sid * ROWS_PER_TILE
    pltpu.sync_copy(u_hbm.at[pl.ds(cid * HALF + l0, ROWS_PER_TILE)],
                    agg_sh.at[pl.ds(l0, ROWS_PER_TILE)])
    pltpu.sync_copy(cnt_hbm.at[cid, sid], cnt_v)
    nb = cnt_v[pl.ds(0, 16)][0]

    plsc.subcore_barrier()

    def _phase2(src_v, sel_v, rows_v):
        def _stage(m, slot):
            pltpu.sync_copy(srcp_hbm.at[cid, sid, pl.ds(m * SMAC, SMAC)],
                            src_v.at[slot])
            pltpu.sync_copy(selp_hbm.at[cid, sid, pl.ds(m * SMAC, SMAC)],
                            sel_v.at[slot])

        def _gather(t, b):
            m = t // SMAC
            pltpu.async_copy(u_hbm.at[src_v.at[m % 2, t % SMAC]],
                             rows_v.at[b], gsem)

        @pl.when(nb > 0)
        def _():
            _stage(0, 0)
            _gather(0, 0)

        @pl.when(nb > 1)
        def _():
            _gather(1, 1)

        @pl.loop(0, nb)
        def _edge(t):
            b = t % 2
            m = t // SMAC
            j = t % SMAC
            pltpu.make_async_copy(u_hbm.at[src_v.at[m % 2, j]],
                                  rows_v.at[b], gsem).wait()
            pltpu.async_copy(rows_v.at[b], agg_sh.at[sel_v.at[m % 2, j]],
                             ssem, add=True)

            @pl.when(jnp.logical_and(j == SMAC - 2, (m + 1) * SMAC < nb))
            def _():
                _stage(m + 1, (m + 1) % 2)

            pltpu.make_async_copy(rows_v.at[b],
                                  agg_sh.at[sel_v.at[m % 2, j]], ssem).wait()

            @pl.when(t + 2 < nb)
            def _():
                _gather(t + 2, b)

    pl.run_scoped(_phase2,
                  pltpu.VMEM((2, SMAC, 128), jnp.int32),
                  pltpu.VMEM((2, SMAC, 128), jnp.int32),
                  pltpu.VMEM((2, 128, F), jnp.float32))

    plsc.subcore_barrier()

    def _phase3(aggb_v, cab_v):
        def _reads(i, b):
            lb = sid * ROWS_PER_TILE + i * RBLK
            pltpu.async_copy(agg_sh.at[pl.ds(lb, RBLK)], aggb_v.at[b], csem)
            pltpu.async_copy(ca_hbm.at[pl.ds(cid * HALF + lb, RBLK)],
                             cab_v.at[b], csem)

        def _wait_reads(i, b):
            lb = sid * ROWS_PER_TILE + i * RBLK
            pltpu.make_async_copy(agg_sh.at[pl.ds(lb, RBLK)], aggb_v.at[b],
                                  csem).wait()
            pltpu.make_async_copy(ca_hbm.at[pl.ds(cid * HALF + lb, RBLK)],
                                  cab_v.at[b], csem).wait()

        def _wait_write(i, b):
            lb = sid * ROWS_PER_TILE + i * RBLK
            pltpu.make_async_copy(aggb_v.at[b],
                                  out_hbm.at[pl.ds(cid * HALF + lb, RBLK)],
                                  wsem).wait()

        @pl.loop(0, NBLK)
        def _combine(i):
            b = i % 2
            lb = sid * ROWS_PER_TILE + i * RBLK
            pltpu.sync_copy(agg_sh.at[pl.ds(lb, RBLK)], aggb_v.at[b])
            pltpu.sync_copy(ca_hbm.at[pl.ds(cid * HALF + lb, RBLK)],
                            cab_v.at[b])

            @pl.loop(0, RBLK)
            def _row(r):
                for q in range(F // 16):
                    sl = pl.ds(q * 16, 16)
                    aggb_v[b, r, sl] = (aggb_v[b, r, sl] * cab_v[b, r, sl]
                                        + cab_v[b, r, pl.ds(F + q * 16, 16)])

            pltpu.sync_copy(aggb_v.at[b],
                            out_hbm.at[pl.ds(cid * HALF + lb, RBLK)])

    pl.run_scoped(_phase3,
                  pltpu.VMEM((2, RBLK, F), jnp.float32),
                  pltpu.VMEM((2, RBLK, 2 * F), jnp.float32))


def _step(u, srcp, selp, cnt, ca):
    return pl.kernel(
        _step_body,
        out_type=jax.ShapeDtypeStruct((NPAD, F), jnp.float32),
        mesh=_mesh,
        scratch_types=[
            pltpu.VMEM((16,), jnp.int32),
            pltpu.VMEM_SHARED((AGG_ROWS, F), jnp.float32),
            pltpu.SemaphoreType.DMA,
            pltpu.SemaphoreType.DMA,
            pltpu.SemaphoreType.DMA,
            pltpu.SemaphoreType.DMA,
        ],
        compiler_params=_sc_params,
    )(u, srcp, selp, cnt, ca)


def kernel(x, edge_index, W1, b1, W2, b2):
    xp = jnp.pad(x, ((0, NPAD - N), (0, 0)))
    h = _mlp(xp, W1, b1, W2, b2)

    src = jnp.pad(edge_index[0], (0, EPAD - E)).reshape(EROWS, 128)
    dst = jnp.pad(edge_index[1], (0, EPAD - E),
                  constant_values=2 ** 20).reshape(EROWS, 128)

    srcp, selp, cnt, deg = _prep(src, dst)
    srcp = srcp.reshape(NC, NS, PROWS, 128)
    selp = selp.reshape(NC, NS, PROWS, 128)
    u, ca1, ca2 = _coeff(deg, h)

    for _ in range(K - 1):
        u = _step(u, srcp, selp, cnt, ca1)
    z = _step(u, srcp, selp, cnt, ca2)
    return z[:N]


# speedup vs baseline: 1.0627x; 1.0127x over previous
import jax
import jax.numpy as jnp
from jax import lax
from jax.experimental import pallas as pl
from jax.experimental.pallas import tpu as pltpu
from jax.experimental.pallas import tpu_sc as plsc

N = 50000
NFEAT = 256
NHID = 256
F = 64
E = 800000
K = 10
ALPHA = 0.1

NC = 2
NS = 16

HALF = 25088
NPAD = 2 * HALF
TRASH = HALF
AGG_ROWS = HALF + 8

ROWS_PER_TILE = HALF // NS
RBLK = 56
NBLK = ROWS_PER_TILE // RBLK

EROWS = 6272
EPAD = EROWS * 128
ROWS_PER_TILE_E = EROWS // NS
MACROS = ROWS_PER_TILE_E // 8

PROWS = 420
SMAC = 14

_mesh = plsc.VectorSubcoreMesh(core_axis_name="c", subcore_axis_name="s",
                               num_cores=NC, num_subcores=NS)
_sc_params = pltpu.CompilerParams(use_tc_tiling_on_sc=False)
_sc_params_nl = pltpu.CompilerParams(use_tc_tiling_on_sc=False,
                                     needs_layout_passes=False)


def _mlp_body(x_ref, w1_ref, b1_ref, w2_ref, b2_ref, o_ref):
    h = jnp.dot(x_ref[...], w1_ref[...], preferred_element_type=jnp.float32)
    h = jnp.maximum(h + b1_ref[...], 0.0)
    o_ref[...] = (
        jnp.dot(h, w2_ref[...], preferred_element_type=jnp.float32)
        + b2_ref[...]
    )


def _mlp(xp, W1, b1, W2, b2):
    blk = 512
    grid = NPAD // blk
    return pl.pallas_call(
        _mlp_body,
        grid=(grid,),
        in_specs=[
            pl.BlockSpec((blk, NFEAT), lambda i: (i, 0)),
            pl.BlockSpec((NFEAT, NHID), lambda i: (0, 0)),
            pl.BlockSpec((1, NHID), lambda i: (0, 0)),
            pl.BlockSpec((NHID, F), lambda i: (0, 0)),
            pl.BlockSpec((1, F), lambda i: (0, 0)),
        ],
        out_specs=pl.BlockSpec((blk, F), lambda i: (i, 0)),
        out_shape=jax.ShapeDtypeStruct((NPAD, F), jnp.float32),
    )(xp, W1, b1.reshape(1, NHID), W2, b2.reshape(1, F))


def _prep_body(src_hbm, dst_hbm, srcp_hbm, selp_hbm, cnt_hbm, deg_hbm,
               dst_v, srcv_v, sel_v, ones_v, degbuf_v, sts_v, stl_v, cnt_v,
               deg_sh, stsem, dsem):
    cid = lax.axis_index("c")
    sid = lax.axis_index("s")

    @pl.loop(0, ROWS_PER_TILE)
    def _zero(i):
        degbuf_v[i, :] = jnp.zeros((16,), jnp.float32)
    pltpu.sync_copy(degbuf_v,
                    deg_sh.at[pl.ds(sid * ROWS_PER_TILE, ROWS_PER_TILE)])

    @pl.loop(0, 128)
    def _ones(i):
        ones_v[i, :] = jnp.ones((16,), jnp.float32)

    plsc.subcore_barrier()

    lo = cid * HALF

    def _stage(g, slot):
        r0 = sid * ROWS_PER_TILE_E + g * 8
        pltpu.async_copy(dst_hbm.at[pl.ds(r0, 8)], dst_v.at[slot], stsem)
        pltpu.async_copy(src_hbm.at[pl.ds(r0, 8)], srcv_v.at[slot], stsem)

    def _stage_wait(g, slot):
        r0 = sid * ROWS_PER_TILE_E + g * 8
        pltpu.make_async_copy(dst_hbm.at[pl.ds(r0, 8)], dst_v.at[slot],
                              stsem).wait()
        pltpu.make_async_copy(src_hbm.at[pl.ds(r0, 8)], srcv_v.at[slot],
                              stsem).wait()

    _stage(0, 0)

    @pl.loop(0, MACROS, init_carry=(jnp.int32(0), jnp.int32(0)))
    def _macro(g, carry):
        off, rows = carry
        p = g % 2
        @pl.when(g > 0)
        def _():
            for j in range(8):
                pltpu.make_async_copy(ones_v, deg_sh.at[sel_v.at[j]],
                                      dsem).wait()

        _stage_wait(g, p)

        @pl.when(g + 1 < MACROS)
        def _():
            _stage(g + 1, 1 - p)

        for j in range(8):
            for q in range(8):
                sl = pl.ds(q * 16, 16)
                d = dst_v[p, j, sl]
                s = srcv_v[p, j, sl]
                loc = d - lo
                ok = (d >= lo) & (d < lo + HALF)
                sel_v[j, sl] = jnp.where(ok, loc, TRASH)
                inc = jnp.where(ok, jnp.int32(1), jnp.int32(0))
                cum = lax.cumsum(inc, axis=0)
                pos = jnp.where(ok, off + cum - 1, jnp.int32(1264))
                plsc.store_scatter(sts_v, [pos], s)
                plsc.store_scatter(stl_v, [pos], loc)
                off = off + cum[15]
                do_flush = off >= 1024

                @pl.when(do_flush)
                def _flush():
                    pltpu.sync_copy(
                        sts_v.at[pl.ds(0, 1024)],
                        srcp_hbm.at[cid, sid, pl.ds(rows * 128, 1024)])
                    pltpu.sync_copy(
                        stl_v.at[pl.ds(0, 1024)],
                        selp_hbm.at[cid, sid, pl.ds(rows * 128, 1024)])
                    sts_v[pl.ds(0, 16)] = sts_v[pl.ds(1024, 16)]
                    stl_v[pl.ds(0, 16)] = stl_v[pl.ds(1024, 16)]

                off = jnp.where(do_flush, off - 1024, off)
                rows = jnp.where(do_flush, rows + 8, rows)
        for j in range(8):
            pltpu.async_copy(ones_v, deg_sh.at[sel_v.at[j]], dsem, add=True)
        return off, rows

    off, rows = _macro
    for j in range(8):
        pltpu.make_async_copy(ones_v, deg_sh.at[sel_v.at[j]], dsem).wait()
    pad_s = jnp.zeros((16,), jnp.int32)
    pad_l = jnp.full((16,), TRASH, jnp.int32)
    sts_v[pl.ds(off, 16)] = pad_s
    stl_v[pl.ds(off, 16)] = pad_l
    target = ((off + 127) // 128) * 128
    for k in range(7):
        pos = off + 16 + k * 16

        @pl.when(pos < target)
        def _pad():
            sts_v[pl.ds(pos, 16)] = pad_s
            stl_v[pl.ds(pos, 16)] = pad_l

    @pl.when(target > 0)
    def _final_flush():
        pltpu.sync_copy(sts_v.at[pl.ds(0, 1152)],
                        srcp_hbm.at[cid, sid, pl.ds(rows * 128, 1152)])
        pltpu.sync_copy(stl_v.at[pl.ds(0, 1152)],
                        selp_hbm.at[cid, sid, pl.ds(rows * 128, 1152)])

    nrows = rows + target // 128
    cnt_v[...] = jnp.full((16,), nrows, jnp.int32)
    pltpu.sync_copy(cnt_v, cnt_hbm.at[cid, sid])

    plsc.subcore_barrier()

    pltpu.sync_copy(deg_sh.at[pl.ds(sid * ROWS_PER_TILE, ROWS_PER_TILE)],
                    deg_hbm.at[pl.ds(cid * HALF + sid * ROWS_PER_TILE,
                                     ROWS_PER_TILE)])


def _prep(src128, dst128):
    return pl.kernel(
        _prep_body,
        out_type=(
            jax.ShapeDtypeStruct((NC, NS, PROWS * 128), jnp.int32),
            jax.ShapeDtypeStruct((NC, NS, PROWS * 128), jnp.int32),
            jax.ShapeDtypeStruct((NC, NS, 16), jnp.int32),
            jax.ShapeDtypeStruct((NPAD, 16), jnp.float32),
        ),
        mesh=_mesh,
        scratch_types=[
            pltpu.VMEM((2, 8, 128), jnp.int32),
            pltpu.VMEM((2, 8, 128), jnp.int32),
            pltpu.VMEM((8, 128), jnp.int32),
            pltpu.VMEM((128, 16), jnp.float32),
            pltpu.VMEM((ROWS_PER_TILE, 16), jnp.float32),
            pltpu.VMEM((1280,), jnp.int32),
            pltpu.VMEM((1280,), jnp.int32),
            pltpu.VMEM((16,), jnp.int32),
            pltpu.VMEM_SHARED((AGG_ROWS, 16), jnp.float32),
            pltpu.SemaphoreType.DMA,
            pltpu.SemaphoreType.DMA,
        ],
        compiler_params=_sc_params_nl,
    )(src128, dst128)


def _coeff_body(deg_ref, h_ref, u_ref, ca1_ref, ca2_ref):
    dinv = lax.rsqrt(deg_ref[:, :1] + 1.0)
    h = h_ref[...]
    u = dinv * h
    u_ref[...] = u
    ca1_ref[...] = jnp.concatenate(
        [jnp.broadcast_to((1.0 - ALPHA) * dinv * dinv, h.shape), ALPHA * u],
        axis=1)
    ca2_ref[...] = jnp.concatenate(
        [jnp.broadcast_to((1.0 - ALPHA) * dinv, h.shape), ALPHA * h], axis=1)


def _coeff(deg, h):
    blk = 512
    grid = NPAD // blk
    o = jax.ShapeDtypeStruct((NPAD, F), jnp.float32)
    o2 = jax.ShapeDtypeStruct((NPAD, 2 * F), jnp.float32)
    return pl.pallas_call(
        _coeff_body,
        grid=(grid,),
        in_specs=[
            pl.BlockSpec((blk, 16), lambda i: (i, 0)),
            pl.BlockSpec((blk, F), lambda i: (i, 0)),
        ],
        out_specs=[
            pl.BlockSpec((blk, F), lambda i: (i, 0)),
            pl.BlockSpec((blk, 2 * F), lambda i: (i, 0)),
            pl.BlockSpec((blk, 2 * F), lambda i: (i, 0)),
        ],
        out_shape=(o, o2, o2),
    )(deg, h)


def _step_body(u_hbm, srcp_hbm, selp_hbm, cnt_hbm, ca_hbm, out_hbm,
               cnt_v, agg_sh, gsem, ssem, csem, wsem):
    cid = lax.axis_index("c")
    sid = lax.axis_index("s")

    l0 = sid * ROWS_PER_TILE
    pltpu.sync_copy(u_hbm.at[pl.ds(cid * HALF + l0, ROWS_PER_TILE)],
                    agg_sh.at[pl.ds(l0, ROWS_PER_TILE)])
    pltpu.sync_copy(cnt_hbm.at[cid, sid], cnt_v)
    nb = cnt_v[pl.ds(0, 16)][0]

    plsc.subcore_barrier()

    def _phase2(src_v, sel_v, rows_v):
        def _stage(m, slot):
            pltpu.sync_copy(srcp_hbm.at[cid, sid, pl.ds(m * SMAC, SMAC)],
                            src_v.at[slot])
            pltpu.sync_copy(selp_hbm.at[cid, sid, pl.ds(m * SMAC, SMAC)],
                            sel_v.at[slot])

        def _gather(t, b):
            m = t // SMAC
            pltpu.async_copy(u_hbm.at[src_v.at[m % 2, t % SMAC]],
                             rows_v.at[b], gsem)

        @pl.when(nb > 0)
        def _():
            _stage(0, 0)
            _gather(0, 0)

        @pl.when(nb > 1)
        def _():
            _gather(1, 1)

        @pl.loop(0, nb)
        def _edge(t):
            b = t % 2
            m = t // SMAC
            j = t % SMAC
            pltpu.make_async_copy(u_hbm.at[src_v.at[m % 2, j]],
                                  rows_v.at[b], gsem).wait()
            pltpu.async_copy(rows_v.at[b], agg_sh.at[sel_v.at[m % 2, j]],
                             ssem, add=True)

            @pl.when(jnp.logical_and(j == SMAC - 2, (m + 1) * SMAC < nb))
            def _():
                _stage(m + 1, (m + 1) % 2)

            pltpu.make_async_copy(rows_v.at[b],
                                  agg_sh.at[sel_v.at[m % 2, j]], ssem).wait()

            @pl.when(t + 2 < nb)
            def _():
                _gather(t + 2, b)

    pl.run_scoped(_phase2,
                  pltpu.VMEM((2, SMAC, 128), jnp.int32),
                  pltpu.VMEM((2, SMAC, 128), jnp.int32),
                  pltpu.VMEM((2, 128, F), jnp.float32))

    plsc.subcore_barrier()

    def _phase3(aggb_v, cab_v):
        def _reads(i, b):
            lb = sid * ROWS_PER_TILE + i * RBLK
            pltpu.async_copy(agg_sh.at[pl.ds(lb, RBLK)], aggb_v.at[b], csem)
            pltpu.async_copy(ca_hbm.at[pl.ds(cid * HALF + lb, RBLK)],
                             cab_v.at[b], csem)

        def _wait_reads(i, b):
            lb = sid * ROWS_PER_TILE + i * RBLK
            pltpu.make_async_copy(agg_sh.at[pl.ds(lb, RBLK)], aggb_v.at[b],
                                  csem).wait()
            pltpu.make_async_copy(ca_hbm.at[pl.ds(cid * HALF + lb, RBLK)],
                                  cab_v.at[b], csem).wait()

        def _wait_write(i, b):
            lb = sid * ROWS_PER_TILE + i * RBLK
            pltpu.make_async_copy(aggb_v.at[b],
                                  out_hbm.at[pl.ds(cid * HALF + lb, RBLK)],
                                  wsem).wait()

        def _wait_write(i):
            b = i % 2
            lb = sid * ROWS_PER_TILE + i * RBLK
            pltpu.make_async_copy(aggb_v.at[b],
                                  out_hbm.at[pl.ds(cid * HALF + lb, RBLK)],
                                  wsem).wait()

        @pl.loop(0, NBLK)
        def _combine(i):
            b = i % 2

            @pl.when(i >= 2)
            def _():
                _wait_write(i - 2)

            lb = sid * ROWS_PER_TILE + i * RBLK
            pltpu.sync_copy(agg_sh.at[pl.ds(lb, RBLK)], aggb_v.at[b])
            pltpu.sync_copy(ca_hbm.at[pl.ds(cid * HALF + lb, RBLK)],
                            cab_v.at[b])

            @pl.loop(0, RBLK)
            def _row(r):
                for q in range(F // 16):
                    sl = pl.ds(q * 16, 16)
                    aggb_v[b, r, sl] = (aggb_v[b, r, sl] * cab_v[b, r, sl]
                                        + cab_v[b, r, pl.ds(F + q * 16, 16)])

            pltpu.async_copy(aggb_v.at[b],
                             out_hbm.at[pl.ds(cid * HALF + lb, RBLK)], wsem)

        _wait_write(NBLK - 2)
        _wait_write(NBLK - 1)

    pl.run_scoped(_phase3,
                  pltpu.VMEM((2, RBLK, F), jnp.float32),
                  pltpu.VMEM((2, RBLK, 2 * F), jnp.float32))


def _step(u, srcp, selp, cnt, ca):
    return pl.kernel(
        _step_body,
        out_type=jax.ShapeDtypeStruct((NPAD, F), jnp.float32),
        mesh=_mesh,
        scratch_types=[
            pltpu.VMEM((16,), jnp.int32),
            pltpu.VMEM_SHARED((AGG_ROWS, F), jnp.float32),
            pltpu.SemaphoreType.DMA,
            pltpu.SemaphoreType.DMA,
            pltpu.SemaphoreType.DMA,
            pltpu.SemaphoreType.DMA,
        ],
        compiler_params=_sc_params,
    )(u, srcp, selp, cnt, ca)


def kernel(x, edge_index, W1, b1, W2, b2):
    xp = jnp.pad(x, ((0, NPAD - N), (0, 0)))
    h = _mlp(xp, W1, b1, W2, b2)

    src = jnp.pad(edge_index[0], (0, EPAD - E)).reshape(EROWS, 128)
    dst = jnp.pad(edge_index[1], (0, EPAD - E),
                  constant_values=2 ** 20).reshape(EROWS, 128)

    srcp, selp, cnt, deg = _prep(src, dst)
    srcp = srcp.reshape(NC, NS, PROWS, 128)
    selp = selp.reshape(NC, NS, PROWS, 128)
    u, ca1, ca2 = _coeff(deg, h)

    for _ in range(K - 1):
        u = _step(u, srcp, selp, cnt, ca1)
    z = _step(u, srcp, selp, cnt, ca2)
    return z[:N]


# speedup vs baseline: 1.1048x; 1.0396x over previous
import jax
import jax.numpy as jnp
from jax import lax
from jax.experimental import pallas as pl
from jax.experimental.pallas import tpu as pltpu
from jax.experimental.pallas import tpu_sc as plsc

N = 50000
NFEAT = 256
NHID = 256
F = 64
E = 800000
K = 10
ALPHA = 0.1

NC = 2
NS = 16

HALF = 25088
NPAD = 2 * HALF
TRASH = HALF
AGG_ROWS = HALF + 8

ROWS_PER_TILE = HALF // NS
RBLK = 56
NBLK = ROWS_PER_TILE // RBLK

EROWS = 6272
EPAD = EROWS * 128
ROWS_PER_TILE_E = EROWS // NS
MACROS = ROWS_PER_TILE_E // 8

PROWS = 420
SMAC = 7

_mesh = plsc.VectorSubcoreMesh(core_axis_name="c", subcore_axis_name="s",
                               num_cores=NC, num_subcores=NS)
_sc_params = pltpu.CompilerParams(use_tc_tiling_on_sc=False)
_sc_params_nl = pltpu.CompilerParams(use_tc_tiling_on_sc=False,
                                     needs_layout_passes=False)


def _mlp_body(x_ref, w1_ref, b1_ref, w2_ref, b2_ref, o_ref):
    h = jnp.dot(x_ref[...], w1_ref[...], preferred_element_type=jnp.float32)
    h = jnp.maximum(h + b1_ref[...], 0.0)
    o_ref[...] = (
        jnp.dot(h, w2_ref[...], preferred_element_type=jnp.float32)
        + b2_ref[...]
    )


def _mlp(xp, W1, b1, W2, b2):
    blk = 512
    grid = NPAD // blk
    return pl.pallas_call(
        _mlp_body,
        grid=(grid,),
        in_specs=[
            pl.BlockSpec((blk, NFEAT), lambda i: (i, 0)),
            pl.BlockSpec((NFEAT, NHID), lambda i: (0, 0)),
            pl.BlockSpec((1, NHID), lambda i: (0, 0)),
            pl.BlockSpec((NHID, F), lambda i: (0, 0)),
            pl.BlockSpec((1, F), lambda i: (0, 0)),
        ],
        out_specs=pl.BlockSpec((blk, F), lambda i: (i, 0)),
        out_shape=jax.ShapeDtypeStruct((NPAD, F), jnp.float32),
    )(xp, W1, b1.reshape(1, NHID), W2, b2.reshape(1, F))


def _prep_body(src_hbm, dst_hbm, srcp_hbm, selp_hbm, cnt_hbm, deg_hbm,
               dst_v, srcv_v, sel_v, ones_v, degbuf_v, sts_v, stl_v, cnt_v,
               deg_sh, stsem, dsem):
    cid = lax.axis_index("c")
    sid = lax.axis_index("s")

    @pl.loop(0, ROWS_PER_TILE)
    def _zero(i):
        degbuf_v[i, :] = jnp.zeros((16,), jnp.float32)
    pltpu.sync_copy(degbuf_v,
                    deg_sh.at[pl.ds(sid * ROWS_PER_TILE, ROWS_PER_TILE)])

    @pl.loop(0, 128)
    def _ones(i):
        ones_v[i, :] = jnp.ones((16,), jnp.float32)

    plsc.subcore_barrier()

    lo = cid * HALF

    def _stage(g, slot):
        r0 = sid * ROWS_PER_TILE_E + g * 8
        pltpu.async_copy(dst_hbm.at[pl.ds(r0, 8)], dst_v.at[slot], stsem)
        pltpu.async_copy(src_hbm.at[pl.ds(r0, 8)], srcv_v.at[slot], stsem)

    def _stage_wait(g, slot):
        r0 = sid * ROWS_PER_TILE_E + g * 8
        pltpu.make_async_copy(dst_hbm.at[pl.ds(r0, 8)], dst_v.at[slot],
                              stsem).wait()
        pltpu.make_async_copy(src_hbm.at[pl.ds(r0, 8)], srcv_v.at[slot],
                              stsem).wait()

    _stage(0, 0)

    @pl.loop(0, MACROS, init_carry=(jnp.int32(0), jnp.int32(0)))
    def _macro(g, carry):
        off, rows = carry
        p = g % 2
        @pl.when(g > 0)
        def _():
            for j in range(8):
                pltpu.make_async_copy(ones_v, deg_sh.at[sel_v.at[j]],
                                      dsem).wait()

        _stage_wait(g, p)

        @pl.when(g + 1 < MACROS)
        def _():
            _stage(g + 1, 1 - p)

        for j in range(8):
            for q in range(8):
                sl = pl.ds(q * 16, 16)
                d = dst_v[p, j, sl]
                s = srcv_v[p, j, sl]
                loc = d - lo
                ok = (d >= lo) & (d < lo + HALF)
                sel_v[j, sl] = jnp.where(ok, loc, TRASH)
                inc = jnp.where(ok, jnp.int32(1), jnp.int32(0))
                cum = lax.cumsum(inc, axis=0)
                pos = jnp.where(ok, off + cum - 1, jnp.int32(1264))
                plsc.store_scatter(sts_v, [pos], s)
                plsc.store_scatter(stl_v, [pos], loc)
                off = off + cum[15]
                do_flush = off >= 1024

                @pl.when(do_flush)
                def _flush():
                    pltpu.sync_copy(
                        sts_v.at[pl.ds(0, 1024)],
                        srcp_hbm.at[cid, sid, pl.ds(rows * 128, 1024)])
                    pltpu.sync_copy(
                        stl_v.at[pl.ds(0, 1024)],
                        selp_hbm.at[cid, sid, pl.ds(rows * 128, 1024)])
                    sts_v[pl.ds(0, 16)] = sts_v[pl.ds(1024, 16)]
                    stl_v[pl.ds(0, 16)] = stl_v[pl.ds(1024, 16)]

                off = jnp.where(do_flush, off - 1024, off)
                rows = jnp.where(do_flush, rows + 8, rows)
        for j in range(8):
            pltpu.async_copy(ones_v, deg_sh.at[sel_v.at[j]], dsem, add=True)
        return off, rows

    off, rows = _macro
    for j in range(8):
        pltpu.make_async_copy(ones_v, deg_sh.at[sel_v.at[j]], dsem).wait()
    pad_s = jnp.zeros((16,), jnp.int32)
    pad_l = jnp.full((16,), TRASH, jnp.int32)
    sts_v[pl.ds(off, 16)] = pad_s
    stl_v[pl.ds(off, 16)] = pad_l
    target = ((off + 127) // 128) * 128
    for k in range(7):
        pos = off + 16 + k * 16

        @pl.when(pos < target)
        def _pad():
            sts_v[pl.ds(pos, 16)] = pad_s
            stl_v[pl.ds(pos, 16)] = pad_l

    @pl.when(target > 0)
    def _final_flush():
        pltpu.sync_copy(sts_v.at[pl.ds(0, 1152)],
                        srcp_hbm.at[cid, sid, pl.ds(rows * 128, 1152)])
        pltpu.sync_copy(stl_v.at[pl.ds(0, 1152)],
                        selp_hbm.at[cid, sid, pl.ds(rows * 128, 1152)])

    nrows = rows + target // 128
    cnt_v[...] = jnp.full((16,), nrows, jnp.int32)
    pltpu.sync_copy(cnt_v, cnt_hbm.at[cid, sid])

    plsc.subcore_barrier()

    pltpu.sync_copy(deg_sh.at[pl.ds(sid * ROWS_PER_TILE, ROWS_PER_TILE)],
                    deg_hbm.at[pl.ds(cid * HALF + sid * ROWS_PER_TILE,
                                     ROWS_PER_TILE)])


def _prep(src128, dst128):
    return pl.kernel(
        _prep_body,
        out_type=(
            jax.ShapeDtypeStruct((NC, NS, PROWS * 128), jnp.int32),
            jax.ShapeDtypeStruct((NC, NS, PROWS * 128), jnp.int32),
            jax.ShapeDtypeStruct((NC, NS, 16), jnp.int32),
            jax.ShapeDtypeStruct((NPAD, 16), jnp.float32),
        ),
        mesh=_mesh,
        scratch_types=[
            pltpu.VMEM((2, 8, 128), jnp.int32),
            pltpu.VMEM((2, 8, 128), jnp.int32),
            pltpu.VMEM((8, 128), jnp.int32),
            pltpu.VMEM((128, 16), jnp.float32),
            pltpu.VMEM((ROWS_PER_TILE, 16), jnp.float32),
            pltpu.VMEM((1280,), jnp.int32),
            pltpu.VMEM((1280,), jnp.int32),
            pltpu.VMEM((16,), jnp.int32),
            pltpu.VMEM_SHARED((AGG_ROWS, 16), jnp.float32),
            pltpu.SemaphoreType.DMA,
            pltpu.SemaphoreType.DMA,
        ],
        compiler_params=_sc_params_nl,
    )(src128, dst128)


def _coeff_body(deg_ref, h_ref, u_ref, ca1_ref, ca2_ref):
    dinv = lax.rsqrt(deg_ref[:, :1] + 1.0)
    h = h_ref[...]
    u = dinv * h
    u_ref[...] = u
    ca1_ref[...] = jnp.concatenate(
        [jnp.broadcast_to((1.0 - ALPHA) * dinv * dinv, h.shape), ALPHA * u],
        axis=1)
    ca2_ref[...] = jnp.concatenate(
        [jnp.broadcast_to((1.0 - ALPHA) * dinv, h.shape), ALPHA * h], axis=1)


def _coeff(deg, h):
    blk = 512
    grid = NPAD // blk
    o = jax.ShapeDtypeStruct((NPAD, F), jnp.float32)
    o2 = jax.ShapeDtypeStruct((NPAD, 2 * F), jnp.float32)
    return pl.pallas_call(
        _coeff_body,
        grid=(grid,),
        in_specs=[
            pl.BlockSpec((blk, 16), lambda i: (i, 0)),
            pl.BlockSpec((blk, F), lambda i: (i, 0)),
        ],
        out_specs=[
            pl.BlockSpec((blk, F), lambda i: (i, 0)),
            pl.BlockSpec((blk, 2 * F), lambda i: (i, 0)),
            pl.BlockSpec((blk, 2 * F), lambda i: (i, 0)),
        ],
        out_shape=(o, o2, o2),
    )(deg, h)


def _step_body(u_hbm, srcp_hbm, selp_hbm, cnt_hbm, ca_hbm, out_hbm,
               cnt_v, agg_sh, gsem, ssem, csem, wsem):
    cid = lax.axis_index("c")
    sid = lax.axis_index("s")

    l0 = sid * ROWS_PER_TILE
    pltpu.sync_copy(u_hbm.at[pl.ds(cid * HALF + l0, ROWS_PER_TILE)],
                    agg_sh.at[pl.ds(l0, ROWS_PER_TILE)])
    pltpu.sync_copy(cnt_hbm.at[cid, sid], cnt_v)
    nb = cnt_v[pl.ds(0, 16)][0]

    plsc.subcore_barrier()

    def _phase2(src_v, sel_v, rows_v):
        def _stage(m, slot):
            pltpu.sync_copy(srcp_hbm.at[cid, sid, pl.ds(m * SMAC, SMAC)],
                            src_v.at[slot])
            pltpu.sync_copy(selp_hbm.at[cid, sid, pl.ds(m * SMAC, SMAC)],
                            sel_v.at[slot])

        def _gather(t, b):
            m = t // SMAC
            pltpu.async_copy(u_hbm.at[src_v.at[m % 2, t % SMAC]],
                             rows_v.at[b], gsem)

        def _drain(t):
            m = t // SMAC
            pltpu.make_async_copy(rows_v.at[t % 3],
                                  agg_sh.at[sel_v.at[m % 2, t % SMAC]],
                                  ssem).wait()

        @pl.when(nb > 0)
        def _():
            _stage(0, 0)
            _gather(0, 0)

        @pl.when(nb > 1)
        def _():
            _gather(1, 1)

        @pl.loop(0, nb)
        def _edge(t):
            b = t % 3
            m = t // SMAC
            j = t % SMAC
            pltpu.make_async_copy(u_hbm.at[src_v.at[m % 2, j]],
                                  rows_v.at[b], gsem).wait()
            pltpu.async_copy(rows_v.at[b], agg_sh.at[sel_v.at[m % 2, j]],
                             ssem, add=True)

            @pl.when(jnp.logical_and(j == SMAC - 2, (m + 1) * SMAC < nb))
            def _():
                _stage(m + 1, (m + 1) % 2)

            @pl.when(t >= 1)
            def _():
                _drain(t - 1)

            @pl.when(t + 2 < nb)
            def _():
                _gather(t + 2, b)

        @pl.when(nb >= 1)
        def _():
            _drain(nb - 1)

    pl.run_scoped(_phase2,
                  pltpu.VMEM((2, SMAC, 128), jnp.int32),
                  pltpu.VMEM((2, SMAC, 128), jnp.int32),
                  pltpu.VMEM((3, 128, F), jnp.float32))

    plsc.subcore_barrier()

    def _phase3(aggb_v, cab_v):
        def _reads(i, b):
            lb = sid * ROWS_PER_TILE + i * RBLK
            pltpu.async_copy(agg_sh.at[pl.ds(lb, RBLK)], aggb_v.at[b], csem)
            pltpu.async_copy(ca_hbm.at[pl.ds(cid * HALF + lb, RBLK)],
                             cab_v.at[b], csem)

        def _wait_reads(i, b):
            lb = sid * ROWS_PER_TILE + i * RBLK
            pltpu.make_async_copy(agg_sh.at[pl.ds(lb, RBLK)], aggb_v.at[b],
                                  csem).wait()
            pltpu.make_async_copy(ca_hbm.at[pl.ds(cid * HALF + lb, RBLK)],
                                  cab_v.at[b], csem).wait()

        def _wait_write(i, b):
            lb = sid * ROWS_PER_TILE + i * RBLK
            pltpu.make_async_copy(aggb_v.at[b],
                                  out_hbm.at[pl.ds(cid * HALF + lb, RBLK)],
                                  wsem).wait()

        def _wait_write(i):
            b = i % 2
            lb = sid * ROWS_PER_TILE + i * RBLK
            pltpu.make_async_copy(aggb_v.at[b],
                                  out_hbm.at[pl.ds(cid * HALF + lb, RBLK)],
                                  wsem).wait()

        @pl.loop(0, NBLK)
        def _combine(i):
            b = i % 2

            @pl.when(i >= 2)
            def _():
                _wait_write(i - 2)

            lb = sid * ROWS_PER_TILE + i * RBLK
            pltpu.sync_copy(agg_sh.at[pl.ds(lb, RBLK)], aggb_v.at[b])
            pltpu.sync_copy(ca_hbm.at[pl.ds(cid * HALF + lb, RBLK)],
                            cab_v.at[b])

            @pl.loop(0, RBLK)
            def _row(r):
                for q in range(F // 16):
                    sl = pl.ds(q * 16, 16)
                    aggb_v[b, r, sl] = (aggb_v[b, r, sl] * cab_v[b, r, sl]
                                        + cab_v[b, r, pl.ds(F + q * 16, 16)])

            pltpu.async_copy(aggb_v.at[b],
                             out_hbm.at[pl.ds(cid * HALF + lb, RBLK)], wsem)

        _wait_write(NBLK - 2)
        _wait_write(NBLK - 1)

    pl.run_scoped(_phase3,
                  pltpu.VMEM((2, RBLK, F), jnp.float32),
                  pltpu.VMEM((2, RBLK, 2 * F), jnp.float32))


def _step(u, srcp, selp, cnt, ca):
    return pl.kernel(
        _step_body,
        out_type=jax.ShapeDtypeStruct((NPAD, F), jnp.float32),
        mesh=_mesh,
        scratch_types=[
            pltpu.VMEM((16,), jnp.int32),
            pltpu.VMEM_SHARED((AGG_ROWS, F), jnp.float32),
            pltpu.SemaphoreType.DMA,
            pltpu.SemaphoreType.DMA,
            pltpu.SemaphoreType.DMA,
            pltpu.SemaphoreType.DMA,
        ],
        compiler_params=_sc_params,
    )(u, srcp, selp, cnt, ca)


def kernel(x, edge_index, W1, b1, W2, b2):
    xp = jnp.pad(x, ((0, NPAD - N), (0, 0)))
    h = _mlp(xp, W1, b1, W2, b2)

    src = jnp.pad(edge_index[0], (0, EPAD - E)).reshape(EROWS, 128)
    dst = jnp.pad(edge_index[1], (0, EPAD - E),
                  constant_values=2 ** 20).reshape(EROWS, 128)

    srcp, selp, cnt, deg = _prep(src, dst)
    srcp = srcp.reshape(NC, NS, PROWS, 128)
    selp = selp.reshape(NC, NS, PROWS, 128)
    u, ca1, ca2 = _coeff(deg, h)

    for _ in range(K - 1):
        u = _step(u, srcp, selp, cnt, ca1)
    z = _step(u, srcp, selp, cnt, ca2)
    return z[:N]
